# baseline (speedup 1.0000x reference)
import functools

import jax
import jax.numpy as jnp
import numpy as np
from jax.experimental import pallas as pl
from jax.experimental.pallas import tpu as pltpu

_N = 10000
_F = 128
_ALPHA = 0.2

_NP = 10240
_BR = 512
_BC = 2560
_PR = 512
_HB = _NP // _BR // 2


def _project_kernel(x_ref, w_ref, b_ref, aw_ref, out_ref, fac_ref):
    i = pl.program_id(0)
    o = jnp.dot(x_ref[...], w_ref[...], preferred_element_type=jnp.float32)
    o = o + b_ref[...]
    row = i * _PR + jax.lax.broadcasted_iota(jnp.int32, (_PR, 1), 0)
    o = jnp.where(row < _N, o, 0.0)
    col = jax.lax.broadcasted_iota(jnp.int32, (_PR, 2 * _F), 1)
    out_ref[...] = jnp.where(col < _F,
                             jnp.pad(o, ((0, 0), (0, _F))),
                             jnp.where(col == _F, 1.0, 0.0))
    st = jnp.dot(o, aw_ref[...], preferred_element_type=jnp.float32)
    stst = jnp.concatenate([st[:, 0:1], _ALPHA * st[:, 0:1],
                            st[:, 1:2], _ALPHA * st[:, 1:2]], axis=1)
    fac_ref[...] = jnp.exp2(stst)


def _ev(adj, es1, es2, et1, et2):
    return jnp.maximum(es1 * et1, es2 * et2) * adj


def _gat_kernel(adjA_ref, adjB_ref, sfA_ref, sfB_ref, tf_ref, out_ref,
                yA_ref, yB_ref, accA_ref, accB_ref, *, nj):
    j = pl.program_id(1)

    @pl.when(j == 0)
    def _init():
        accA_ref[...] = jnp.zeros_like(accA_ref)
        accB_ref[...] = jnp.zeros_like(accB_ref)

    et1 = tf_ref[0:1, pl.ds(j * _BC, _BC)]
    et2 = tf_ref[1:2, pl.ds(j * _BC, _BC)]
    rhs = out_ref[pl.ds(j * _BC, _BC), :]
    evA = _ev(adjA_ref[...], sfA_ref[:, 0:1], sfA_ref[:, 1:2], et1, et2)
    evB = _ev(adjB_ref[...], sfB_ref[:, 0:1], sfB_ref[:, 1:2], et1, et2)

    @pl.when(j < nj - 1)
    def _acc_body():
        accA_ref[...] += jnp.dot(evA, rhs, preferred_element_type=jnp.float32)
        accB_ref[...] += jnp.dot(evB, rhs, preferred_element_type=jnp.float32)

    @pl.when(j == nj - 1)
    def _acc_last():
        col = j * _BC + jax.lax.broadcasted_iota(jnp.int32, (_BR, _BC), 1)
        mask = col < _N
        accA = accA_ref[...] + jnp.dot(jnp.where(mask, evA, 0.0), rhs,
                                       preferred_element_type=jnp.float32)
        accB = accB_ref[...] + jnp.dot(jnp.where(mask, evB, 0.0), rhs,
                                       preferred_element_type=jnp.float32)
        rsA = accA[:, _F:_F + 1]
        rsB = accB[:, _F:_F + 1]
        yA_ref[...] = accA[:, :_F] / jnp.where(rsA == 0.0, 1.0, rsA)
        yB_ref[...] = accB[:, :_F] / jnp.where(rsB == 0.0, 1.0, rsB)


def kernel(input, adj, W, b, attn_w):
    aw = attn_w.reshape(_F, 2) * np.float32(np.log2(np.e))
    b2 = b.reshape(1, _F)

    out, fac = pl.pallas_call(
        _project_kernel,
        grid=(_NP // _PR,),
        in_specs=[
            pl.BlockSpec((_PR, _F), lambda i: (i, 0)),
            pl.BlockSpec((_F, _F), lambda i: (0, 0)),
            pl.BlockSpec((1, _F), lambda i: (0, 0)),
            pl.BlockSpec((_F, 2), lambda i: (0, 0)),
        ],
        out_specs=[
            pl.BlockSpec((_PR, 2 * _F), lambda i: (i, 0)),
            pl.BlockSpec((_PR, 4), lambda i: (i, 0)),
        ],
        out_shape=[
            jax.ShapeDtypeStruct((_NP, 2 * _F), jnp.float32),
            jax.ShapeDtypeStruct((_NP, 4), jnp.float32),
        ],
    )(input, W, b2, aw)

    sf = fac[:, 0:2]
    tf = fac[:, 2:4].T

    nhalf = _NP // 2
    ni, nj = _HB, _NP // _BC
    yA, yB = pl.pallas_call(
        functools.partial(_gat_kernel, nj=nj),
        grid=(ni, nj),
        in_specs=[
            pl.BlockSpec((_BR, _BC), lambda i, j: (i, j)),
            pl.BlockSpec((_BR, _BC), lambda i, j: (i + _HB, j)),
            pl.BlockSpec((_BR, 2), lambda i, j: (i, 0)),
            pl.BlockSpec((_BR, 2), lambda i, j: (i + _HB, 0)),
            pl.BlockSpec((2, _NP), lambda i, j: (0, 0)),
            pl.BlockSpec((_NP, 2 * _F), lambda i, j: (0, 0)),
        ],
        out_specs=[
            pl.BlockSpec((_BR, _F), lambda i, j: (i, 0)),
            pl.BlockSpec((_BR, _F), lambda i, j: (i, 0)),
        ],
        out_shape=[
            jax.ShapeDtypeStruct((nhalf, _F), jnp.float32),
            jax.ShapeDtypeStruct((_N - nhalf, _F), jnp.float32),
        ],
        scratch_shapes=[
            pltpu.VMEM((_BR, 2 * _F), jnp.float32),
            pltpu.VMEM((_BR, 2 * _F), jnp.float32),
        ],
    )(adj, adj, sf, sf, tf, out)

    return jnp.concatenate([yA, yB], axis=0)


# speedup vs baseline: 1.1015x; 1.1015x over previous
import functools

import jax
import jax.numpy as jnp
import numpy as np
from jax.experimental import pallas as pl
from jax.experimental.pallas import tpu as pltpu

_N = 10000
_F = 128
_ALPHA = 0.2

_NP = 10240
_BR = 512
_BC = 2560
_PR = 512
_HB = _NP // _BR // 2


def _project_kernel(x_ref, w_ref, b_ref, aw_ref, out_ref, fac_ref):
    i = pl.program_id(0)
    o = jnp.dot(x_ref[...], w_ref[...], preferred_element_type=jnp.float32)
    o = o + b_ref[...]
    row = i * _PR + jax.lax.broadcasted_iota(jnp.int32, (_PR, 1), 0)
    o = jnp.where(row < _N, o, 0.0)
    col = jax.lax.broadcasted_iota(jnp.int32, (_PR, 2 * _F), 1)
    out_ref[...] = jnp.where(col < _F,
                             jnp.pad(o, ((0, 0), (0, _F))),
                             jnp.where(col == _F, 1.0, 0.0)).astype(jnp.bfloat16)
    st = jnp.dot(o, aw_ref[...], preferred_element_type=jnp.float32)
    stst = jnp.concatenate([(_ALPHA - 1.0) * st[:, 0:1],
                            st[:, 1:2], _ALPHA * st[:, 1:2],
                            jnp.zeros_like(st[:, 0:1])], axis=1)
    fac_ref[...] = jnp.exp2(stst).astype(jnp.bfloat16)


def _ev(adj, r, et1, et2):
    return jnp.maximum(et1, r * et2) * adj.astype(jnp.bfloat16)


def _gat_kernel(adjA_ref, adjB_ref, sfA_ref, sfB_ref, tf_ref, out_ref,
                yA_ref, yB_ref, accA_ref, accB_ref, *, nj):
    j = pl.program_id(1)

    @pl.when(j == 0)
    def _init():
        accA_ref[...] = jnp.zeros_like(accA_ref)
        accB_ref[...] = jnp.zeros_like(accB_ref)

    et1 = tf_ref[0:1, pl.ds(j * _BC, _BC)]
    et2 = tf_ref[1:2, pl.ds(j * _BC, _BC)]
    rhs = out_ref[pl.ds(j * _BC, _BC), :]
    evA = _ev(adjA_ref[...], sfA_ref[:, 0:1], et1, et2)
    evB = _ev(adjB_ref[...], sfB_ref[:, 0:1], et1, et2)

    @pl.when(j < nj - 1)
    def _acc_body():
        accA_ref[...] += jnp.dot(evA, rhs, preferred_element_type=jnp.float32)
        accB_ref[...] += jnp.dot(evB, rhs, preferred_element_type=jnp.float32)

    @pl.when(j == nj - 1)
    def _acc_last():
        col = j * _BC + jax.lax.broadcasted_iota(jnp.int32, (_BR, _BC), 1)
        mask = col < _N
        accA = accA_ref[...] + jnp.dot(jnp.where(mask, evA, jnp.bfloat16(0.0)), rhs,
                                       preferred_element_type=jnp.float32)
        accB = accB_ref[...] + jnp.dot(jnp.where(mask, evB, jnp.bfloat16(0.0)), rhs,
                                       preferred_element_type=jnp.float32)
        rsA = accA[:, _F:_F + 1]
        rsB = accB[:, _F:_F + 1]
        yA_ref[...] = accA[:, :_F] / jnp.where(rsA == 0.0, 1.0, rsA)
        yB_ref[...] = accB[:, :_F] / jnp.where(rsB == 0.0, 1.0, rsB)


def kernel(input, adj, W, b, attn_w):
    aw = attn_w.reshape(_F, 2) * np.float32(np.log2(np.e))
    b2 = b.reshape(1, _F)

    out, fac = pl.pallas_call(
        _project_kernel,
        grid=(_NP // _PR,),
        in_specs=[
            pl.BlockSpec((_PR, _F), lambda i: (i, 0)),
            pl.BlockSpec((_F, _F), lambda i: (0, 0)),
            pl.BlockSpec((1, _F), lambda i: (0, 0)),
            pl.BlockSpec((_F, 2), lambda i: (0, 0)),
        ],
        out_specs=[
            pl.BlockSpec((_PR, 2 * _F), lambda i: (i, 0)),
            pl.BlockSpec((_PR, 4), lambda i: (i, 0)),
        ],
        out_shape=[
            jax.ShapeDtypeStruct((_NP, 2 * _F), jnp.bfloat16),
            jax.ShapeDtypeStruct((_NP, 4), jnp.bfloat16),
        ],
    )(input, W, b2, aw)

    sf = fac[:, 0:2]
    tf = fac[:, 1:3].T

    nhalf = _NP // 2
    ni, nj = _HB, _NP // _BC
    yA, yB = pl.pallas_call(
        functools.partial(_gat_kernel, nj=nj),
        grid=(ni, nj),
        in_specs=[
            pl.BlockSpec((_BR, _BC), lambda i, j: (i, j)),
            pl.BlockSpec((_BR, _BC), lambda i, j: (i + _HB, j)),
            pl.BlockSpec((_BR, 2), lambda i, j: (i, 0)),
            pl.BlockSpec((_BR, 2), lambda i, j: (i + _HB, 0)),
            pl.BlockSpec((2, _NP), lambda i, j: (0, 0)),
            pl.BlockSpec((_NP, 2 * _F), lambda i, j: (0, 0)),
        ],
        out_specs=[
            pl.BlockSpec((_BR, _F), lambda i, j: (i, 0)),
            pl.BlockSpec((_BR, _F), lambda i, j: (i, 0)),
        ],
        out_shape=[
            jax.ShapeDtypeStruct((nhalf, _F), jnp.float32),
            jax.ShapeDtypeStruct((_N - nhalf, _F), jnp.float32),
        ],
        scratch_shapes=[
            pltpu.VMEM((_BR, 2 * _F), jnp.float32),
            pltpu.VMEM((_BR, 2 * _F), jnp.float32),
        ],
    )(adj, adj, sf, sf, tf, out)

    return jnp.concatenate([yA, yB], axis=0)
